# Initial kernel scaffold; baseline (speedup 1.0000x reference)
import jax
import jax.numpy as jnp
from jax.experimental import pallas as pl


def kernel(input_ids, word_embeddings):
    raise NotImplementedError("write your pallas kernel here")


# speedup vs baseline: 1.7032x; 1.7032x over previous
import functools

import jax
import jax.numpy as jnp
from jax import lax
from jax.experimental import pallas as pl
from jax.experimental.pallas import tpu as pltpu
from jax.experimental.pallas import tpu_sc as plsc

VOCAB = 150528
HIDDEN = 12288
TOKENS = 8192

NC, NS = 2, 16
NW = NC * NS
ROWS_PER_W = TOKENS // NW
K = 4
CH = ROWS_PER_W // K

_mesh = plsc.VectorSubcoreMesh(
    core_axis_name="c", subcore_axis_name="s", num_cores=NC, num_subcores=NS
)


@functools.partial(
    pl.kernel,
    mesh=_mesh,
    out_type=jax.ShapeDtypeStruct((TOKENS, HIDDEN), jnp.float32),
    scratch_types=[
        pltpu.VMEM((CH, K), jnp.int32),
        pltpu.VMEM((K, HIDDEN), jnp.float32),
        pltpu.SemaphoreType.DMA,
    ],
)
def _sc_gather(idx_hbm, table_hbm, out_hbm, idx_v, buf, sem):
    wid = lax.axis_index("s") * NC + lax.axis_index("c")
    base = wid * ROWS_PER_W
    pltpu.sync_copy(idx_hbm.at[wid], idx_v)

    def body(ci, carry):
        pltpu.async_copy(table_hbm.at[idx_v.at[ci]], buf, sem).wait()
        pltpu.sync_copy(buf, out_hbm.at[pl.ds(base + ci * K, K)])
        return carry

    lax.fori_loop(0, CH, body, 0)


def kernel(input_ids, word_embeddings):
    b, s = input_ids.shape
    idx = input_ids.reshape(NW, CH, K)
    out = _sc_gather(idx, word_embeddings)
    return out.reshape(b, s, HIDDEN)


# speedup vs baseline: 1.9046x; 1.1183x over previous
import functools

import jax
import jax.numpy as jnp
from jax import lax
from jax.experimental import pallas as pl
from jax.experimental.pallas import tpu as pltpu
from jax.experimental.pallas import tpu_sc as plsc

VOCAB = 150528
HIDDEN = 12288
TOKENS = 8192

NC, NS = 2, 16
NW = NC * NS
ROWS_PER_W = TOKENS // NW
K = 4
CH = ROWS_PER_W // K
G = CH // 2

_mesh = plsc.VectorSubcoreMesh(
    core_axis_name="c", subcore_axis_name="s", num_cores=NC, num_subcores=NS
)


@functools.partial(
    pl.kernel,
    mesh=_mesh,
    out_type=jax.ShapeDtypeStruct((TOKENS, HIDDEN), jnp.float32),
    scratch_types=[
        pltpu.VMEM((CH, K), jnp.int32),
        pltpu.VMEM((K, HIDDEN), jnp.float32),
        pltpu.VMEM((K, HIDDEN), jnp.float32),
        pltpu.SemaphoreType.DMA,
        pltpu.SemaphoreType.DMA,
        pltpu.SemaphoreType.DMA,
        pltpu.SemaphoreType.DMA,
    ],
)
def _sc_gather(idx_hbm, table_hbm, out_hbm, idx_v, buf0, buf1, g0, g1, w0, w1):
    wid = lax.axis_index("s") * NC + lax.axis_index("c")
    base = wid * ROWS_PER_W
    pltpu.sync_copy(idx_hbm.at[wid], idx_v)

    bufs = (buf0, buf1)
    gsem = (g0, g1)
    wsem = (w0, w1)

    def gather_desc(c, b):
        return pltpu.make_async_copy(table_hbm.at[idx_v.at[c]], bufs[b], gsem[b])

    def write_desc(c, b):
        return pltpu.make_async_copy(
            bufs[b], out_hbm.at[pl.ds(base + c * K, K)], wsem[b]
        )

    gather_desc(0, 0).start()
    gather_desc(1, 1).start()

    def body(g, carry):
        for b in range(2):
            c = 2 * g + b
            gather_desc(c, b).wait()
            write_desc(c, b).start()
        for b in range(2):
            c = 2 * g + b
            write_desc(c, b).wait()
            gather_desc(c + 2, b).start()
        return carry

    lax.fori_loop(0, G - 1, body, 0)

    for b in range(2):
        c = CH - 2 + b
        gather_desc(c, b).wait()
        write_desc(c, b).start()
    for b in range(2):
        write_desc(CH - 2 + b, b).wait()


def kernel(input_ids, word_embeddings):
    b, s = input_ids.shape
    idx = input_ids.reshape(NW, CH, K)
    out = _sc_gather(idx, word_embeddings)
    return out.reshape(b, s, HIDDEN)
